# Initial kernel scaffold; baseline (speedup 1.0000x reference)
import jax
import jax.numpy as jnp
from jax.experimental import pallas as pl


def kernel(x, W_all, b_all, W_fk, b_fk, W_rk, b_rk, neuron_emb):
    raise NotImplementedError("write your pallas kernel here")


# speedup vs baseline: 1.3016x; 1.3016x over previous
import jax
import jax.numpy as jnp
from jax.experimental import pallas as pl

D_MODEL = 2048
D_SPACE = 64
_POOL_SIZES = (1024, 1024, 1024, 1024, 1024, 1024, 4096, 4096)
_TOTAL = sum(_POOL_SIZES)
_BT = 256


def _body(x_ref, w_ref, b_ref, emb_ref, *out_refs):
    proj = jnp.dot(x_ref[...], w_ref[...], preferred_element_type=jnp.float32)
    proj = proj + b_ref[...]
    start = 0
    for i, (n, o_ref) in enumerate(zip(_POOL_SIZES, out_refs)):
        h = proj[:, i * D_SPACE:(i + 1) * D_SPACE]
        e = emb_ref[start:start + n, :]
        normsq = jnp.sum(e * e, axis=1, keepdims=True)
        inv = 1.0 / jnp.maximum(jnp.sqrt(normsq), 1e-12)
        en = e * inv
        o_ref[...] = jax.lax.dot_general(
            h, en, (((1,), (1,)), ((), ())),
            preferred_element_type=jnp.float32)
        start += n


def kernel(x, W_all, b_all, W_fk, b_fk, W_rk, b_rk, neuron_emb):
    B, S, _ = x.shape
    T = B * S
    x2 = x.reshape(T, D_MODEL)
    W = jnp.concatenate([W_all, W_fk, W_rk], axis=1)
    b = jnp.concatenate([b_all, b_fk, b_rk]).reshape(1, 8 * D_SPACE)

    n_blocks = T // _BT
    full = lambda i: (0, 0)
    out_shapes = [jax.ShapeDtypeStruct((T, n), jnp.float32) for n in _POOL_SIZES]
    out_specs = [pl.BlockSpec((_BT, n), lambda i: (i, 0)) for n in _POOL_SIZES]

    outs = pl.pallas_call(
        _body,
        grid=(n_blocks,),
        in_specs=[
            pl.BlockSpec((_BT, D_MODEL), lambda i: (i, 0)),
            pl.BlockSpec((D_MODEL, 8 * D_SPACE), full),
            pl.BlockSpec((1, 8 * D_SPACE), full),
            pl.BlockSpec((_TOTAL, D_SPACE), full),
        ],
        out_specs=out_specs,
        out_shape=out_shapes,
    )(x2, W, b, neuron_emb)

    return tuple(o.reshape(B, S, n) for o, n in zip(outs, _POOL_SIZES))
